# baseline (speedup 1.0000x reference)
import jax
import jax.numpy as jnp
from jax.experimental import pallas as pl
from jax.experimental.pallas import tpu as pltpu

_K = 16


def _kan_body(grid_ref, x_ref, v_ref, sw_ref, sb_ref, o_ref):
    B, D = x_ref.shape
    DK = D * _K
    xc = jnp.clip(x_ref[...], -1.0, 1.0)
    g0 = grid_ref[0]
    inv_h = (_K - 1) / (grid_ref[_K - 1] - g0)
    r = jax.lax.broadcasted_iota(jnp.int32, (D, DK), 0)
    c = jax.lax.broadcasted_iota(jnp.int32, (D, DK), 1)
    E = jnp.where(r == c // _K, 1.0, 0.0)
    xrep = jax.lax.dot(xc, E, preferred_element_type=jnp.float32)
    kk = jax.lax.broadcasted_iota(jnp.int32, (1, DK), 1) % _K
    gt = jnp.zeros((1, DK), jnp.float32)
    for k in range(_K):
        gt = jnp.where(kk == k, grid_ref[k] * inv_h, gt)
    S = jnp.maximum(1.0 - jnp.abs(xrep * inv_h - gt), 0.0)
    acc = jax.lax.dot_general(S, v_ref[...], (((1,), (1,)), ((), ())),
                              preferred_element_type=jnp.float32)
    acc = acc + jax.lax.dot_general(xc, sw_ref[...], (((1,), (1,)), ((), ())),
                                    preferred_element_type=jnp.float32)
    o_ref[...] = acc + sb_ref[...]


def kernel(x, values, skip_w, skip_b, grid):
    B, D = x.shape
    O = values.shape[0]
    vr = values.reshape(O, D * _K)
    sb = skip_b.reshape(1, O)
    return pl.pallas_call(
        _kan_body,
        out_shape=jax.ShapeDtypeStruct((B, O), jnp.float32),
        in_specs=[pl.BlockSpec(memory_space=pltpu.SMEM)]
        + [pl.BlockSpec(memory_space=pltpu.VMEM)] * 4,
        out_specs=pl.BlockSpec(memory_space=pltpu.VMEM),
    )(grid, x, vr, skip_w, sb)
